# Initial kernel scaffold; baseline (speedup 1.0000x reference)
import jax
import jax.numpy as jnp
from jax.experimental import pallas as pl


def kernel(x, edge_index, W, b):
    raise NotImplementedError("write your pallas kernel here")


# speedup vs baseline: 20.3212x; 20.3212x over previous
import functools

import jax
import jax.numpy as jnp
from jax import lax
from jax.experimental import pallas as pl
from jax.experimental.pallas import tpu as pltpu
from jax.experimental.pallas import tpu_sc as plsc

N = 10000
E = 320000
D = 128

NC = 2
NS = 16
NW = NC * NS
CHUNK = 128
CPT = 79
EPT = CPT * CHUNK
E_PAD = NW * EPT
N_ACC = 10240
RPT = N_ACC // NS
DUMMY = N

_mesh = plsc.VectorSubcoreMesh(core_axis_name="c", subcore_axis_name="s")


@functools.partial(
    pl.kernel,
    out_type=jax.ShapeDtypeStruct((NC, N_ACC), jnp.float32),
    mesh=_mesh,
    scratch_types=[
        pltpu.VMEM((CPT, CHUNK), jnp.int32),
        pltpu.VMEM((CHUNK,), jnp.float32),
        pltpu.VMEM((RPT,), jnp.float32),
        pltpu.VMEM_SHARED((N_ACC,), jnp.float32),
    ],
)
def _deg_kernel(dst_hbm, out_hbm, idx_v, ones_v, zeros_v, hist_sh):
    c = lax.axis_index("c")
    s = lax.axis_index("s")
    wid = c * NS + s

    @pl.loop(0, CHUNK, step=16)
    def _(i):
        ones_v[pl.ds(i, 16)] = jnp.full((16,), 1.0, jnp.float32)

    @pl.loop(0, RPT, step=16)
    def _(i):
        zeros_v[pl.ds(i, 16)] = jnp.zeros((16,), jnp.float32)

    pltpu.sync_copy(zeros_v, hist_sh.at[pl.ds(s * RPT, RPT)])
    plsc.subcore_barrier()

    pltpu.sync_copy(dst_hbm.at[wid], idx_v)

    @pl.loop(0, CPT)
    def _(j):
        pltpu.sync_copy(ones_v, hist_sh.at[idx_v.at[j]], add=True)

    plsc.subcore_barrier()
    pltpu.sync_copy(
        hist_sh.at[pl.ds(s * RPT, RPT)], out_hbm.at[c, pl.ds(s * RPT, RPT)]
    )


@functools.partial(
    pl.kernel,
    out_type=jax.ShapeDtypeStruct((NC, N_ACC, D), jnp.float32),
    mesh=_mesh,
    scratch_types=[
        pltpu.VMEM((CPT, CHUNK), jnp.int32),
        pltpu.VMEM((CPT, CHUNK), jnp.int32),
        pltpu.VMEM((CHUNK, D), jnp.float32),
        pltpu.VMEM_SHARED((N_ACC, D), jnp.float32),
    ],
)
def _edge_kernel(src_hbm, dst_hbm, g_hbm, out_hbm, si_v, di_v, buf_v, acc_sh):
    c = lax.axis_index("c")
    s = lax.axis_index("s")
    wid = c * NS + s

    @pl.loop(0, CHUNK)
    def _(i):
        @pl.loop(0, D, step=16)
        def _(k):
            buf_v[i, pl.ds(k, 16)] = jnp.zeros((16,), jnp.float32)

    @pl.loop(0, RPT, step=CHUNK)
    def _(r):
        pltpu.sync_copy(buf_v, acc_sh.at[pl.ds(s * RPT + r, CHUNK)])

    plsc.subcore_barrier()

    pltpu.sync_copy(src_hbm.at[wid], si_v)
    pltpu.sync_copy(dst_hbm.at[wid], di_v)

    @pl.loop(0, CPT)
    def _(j):
        pltpu.sync_copy(g_hbm.at[si_v.at[j]], buf_v)
        pltpu.sync_copy(buf_v, acc_sh.at[di_v.at[j]], add=True)

    plsc.subcore_barrier()
    pltpu.sync_copy(
        acc_sh.at[pl.ds(s * RPT, RPT)], out_hbm.at[c, pl.ds(s * RPT, RPT)]
    )


BLK = 1024


def _mm_body(hist_ref, x_ref, w_ref, g_ref, dinv_ref):
    deg = hist_ref[0, :] + hist_ref[1, :] + 1.0
    dinv = lax.rsqrt(deg)
    h = jnp.dot(x_ref[...], w_ref[...], preferred_element_type=jnp.float32)
    g_ref[...] = h * dinv[:, None]
    dinv_ref[...] = dinv[None, :]


def _fin_body(acc_ref, g_ref, dinv_ref, b_ref, o_ref):
    t = (acc_ref[0] + acc_ref[1] + g_ref[...]) * dinv_ref[0, :][:, None]
    t = t + b_ref[0, :][None, :]
    o_ref[...] = t * jax.nn.sigmoid(t)


@jax.jit
def kernel(x, edge_index, W, b):
    src = edge_index[0]
    dst = edge_index[1]
    pad = E_PAD - E
    src_p = jnp.concatenate(
        [src, jnp.zeros((pad,), jnp.int32)]
    ).reshape(NW, CPT, CHUNK)
    dst_p = jnp.concatenate(
        [dst, jnp.full((pad,), DUMMY, jnp.int32)]
    ).reshape(NW, CPT, CHUNK)

    hist = _deg_kernel(dst_p)

    x_p = jnp.concatenate([x, jnp.zeros((N_ACC - N, D), x.dtype)], axis=0)

    g, dinv = pl.pallas_call(
        _mm_body,
        grid=(N_ACC // BLK,),
        in_specs=[
            pl.BlockSpec((2, BLK), lambda i: (0, i)),
            pl.BlockSpec((BLK, D), lambda i: (i, 0)),
            pl.BlockSpec((D, D), lambda i: (0, 0)),
        ],
        out_specs=[
            pl.BlockSpec((BLK, D), lambda i: (i, 0)),
            pl.BlockSpec((1, BLK), lambda i: (0, i)),
        ],
        out_shape=[
            jax.ShapeDtypeStruct((N_ACC, D), jnp.float32),
            jax.ShapeDtypeStruct((1, N_ACC), jnp.float32),
        ],
    )(hist, x_p, W)

    acc = _edge_kernel(src_p, dst_p, g)

    out = pl.pallas_call(
        _fin_body,
        grid=(N_ACC // BLK,),
        in_specs=[
            pl.BlockSpec((2, BLK, D), lambda i: (0, i, 0)),
            pl.BlockSpec((BLK, D), lambda i: (i, 0)),
            pl.BlockSpec((1, BLK), lambda i: (0, i)),
            pl.BlockSpec((1, D), lambda i: (0, 0)),
        ],
        out_specs=pl.BlockSpec((BLK, D), lambda i: (i, 0)),
        out_shape=jax.ShapeDtypeStruct((N_ACC, D), jnp.float32),
    )(acc, g, dinv, b.reshape(1, D))

    return out[:N]
